# Initial kernel scaffold; baseline (speedup 1.0000x reference)
import jax
import jax.numpy as jnp
from jax.experimental import pallas as pl


def kernel(tokens, table):
    raise NotImplementedError("write your pallas kernel here")


# speedup vs baseline: 4.7371x; 4.7371x over previous
import math

import jax
import jax.numpy as jnp
from jax import lax
from jax.experimental import pallas as pl
from jax.experimental.pallas import tpu as pltpu
from jax.experimental.pallas import tpu_sc as plsc

D = 128
NC, NS = 2, 16
NW = NC * NS
CG = 128
LANES = 16
SCALE = math.sqrt(128.0)


def _body(tok_hbm, table_hbm, out_hbm, idx_v, gbuf, gsem):
    wid = lax.axis_index("s") * NC + lax.axis_index("c")
    ng = idx_v.shape[0]
    pltpu.sync_copy(tok_hbm.at[wid], idx_v)

    def chunk(g, carry):
        pltpu.async_copy(table_hbm.at[idx_v.at[g]], gbuf, gsem).wait()

        def row(r, c):
            for j in range(D // LANES):
                sl = pl.ds(LANES * j, LANES)
                gbuf[r, sl] = gbuf[r, sl] * SCALE
            return c

        lax.fori_loop(0, CG, row, 0)
        pltpu.sync_copy(gbuf, out_hbm.at[wid, g])
        return carry

    lax.fori_loop(0, ng, chunk, 0)


def kernel(tokens, table):
    b0, b1 = tokens.shape
    ng = (b0 * b1) // (NW * CG)
    tok = tokens.reshape(NW, ng, CG).astype(jnp.int32)
    out = pl.kernel(
        _body,
        out_type=jax.ShapeDtypeStruct((NW, ng, CG, D), jnp.float32),
        mesh=plsc.VectorSubcoreMesh(core_axis_name="c", subcore_axis_name="s"),
        scratch_types=[
            pltpu.VMEM((ng, CG), jnp.int32),
            pltpu.VMEM((CG, D), jnp.float32),
            pltpu.SemaphoreType.DMA,
        ],
    )(tok, table)
    return out.reshape(b0, b1, D)


# speedup vs baseline: 7.8561x; 1.6584x over previous
import math

import jax
import jax.numpy as jnp
from jax import lax
from jax.experimental import pallas as pl
from jax.experimental.pallas import tpu as pltpu
from jax.experimental.pallas import tpu_sc as plsc

D = 128
NC, NS = 2, 16
NW = NC * NS
CG = 128
LANES = 16
SCALE = math.sqrt(128.0)


def _body(tok_hbm, table_hbm, out_hbm,
          idx_v, gbuf0, gbuf1, obuf0, obuf1, gsem0, gsem1, osem0, osem1):
    wid = lax.axis_index("s") * NC + lax.axis_index("c")
    ng = idx_v.shape[0]
    bufs = ((gbuf0, obuf0, gsem0, osem0), (gbuf1, obuf1, gsem1, osem1))
    pltpu.sync_copy(tok_hbm.at[wid], idx_v)

    pltpu.async_copy(table_hbm.at[idx_v.at[0]], gbuf0, gsem0)
    pltpu.async_copy(table_hbm.at[idx_v.at[1]], gbuf1, gsem1)

    def outer(k, carry):
        for b, (gbuf, obuf, gsem, osem) in enumerate(bufs):
            g = 2 * k + b
            pltpu.make_async_copy(table_hbm.at[idx_v.at[g]], gbuf, gsem).wait()

            @pl.when(k > 0)
            def _():
                pltpu.make_async_copy(obuf, out_hbm.at[wid, g], osem).wait()

            def row(r, c):
                for j in range(D // LANES):
                    sl = pl.ds(LANES * j, LANES)
                    obuf[r, sl] = gbuf[r, sl] * SCALE
                return c

            lax.fori_loop(0, CG, row, 0)

            @pl.when(k < ng // 2 - 1)
            def _():
                pltpu.async_copy(table_hbm.at[idx_v.at[g + 2]], gbuf, gsem)

            pltpu.async_copy(obuf, out_hbm.at[wid, g], osem)
        return carry

    lax.fori_loop(0, ng // 2, outer, 0)
    pltpu.make_async_copy(obuf0, out_hbm.at[wid, ng - 2], osem0).wait()
    pltpu.make_async_copy(obuf1, out_hbm.at[wid, ng - 1], osem1).wait()


def kernel(tokens, table):
    b0, b1 = tokens.shape
    ng = (b0 * b1) // (NW * CG)
    tok = tokens.reshape(NW, ng, CG).astype(jnp.int32)
    out = pl.kernel(
        _body,
        out_type=jax.ShapeDtypeStruct((NW, ng, CG, D), jnp.float32),
        mesh=plsc.VectorSubcoreMesh(core_axis_name="c", subcore_axis_name="s"),
        scratch_types=[
            pltpu.VMEM((ng, CG), jnp.int32),
            pltpu.VMEM((CG, D), jnp.float32),
            pltpu.VMEM((CG, D), jnp.float32),
            pltpu.VMEM((CG, D), jnp.float32),
            pltpu.VMEM((CG, D), jnp.float32),
            pltpu.SemaphoreType.DMA,
            pltpu.SemaphoreType.DMA,
            pltpu.SemaphoreType.DMA,
            pltpu.SemaphoreType.DMA,
        ],
    )(tok, table)
    return out.reshape(b0, b1, D)
